# baseline (speedup 1.0000x reference)
import functools

import jax
import jax.numpy as jnp
from jax import lax
from jax.experimental import pallas as pl
from jax.experimental.pallas import tpu as pltpu
from jax.experimental.pallas import tpu_sc as plsc

_NC = 2
_NS = 16
_NW = _NC * _NS
_LANES = 16


@functools.cache
def _make_kernel(batch, seq, emb):
    blk = batch // _NW
    assert blk == 128 and emb % 8 == 0 and seq % 2 == 0
    te = emb // 8
    mesh = plsc.VectorSubcoreMesh(core_axis_name="c", subcore_axis_name="s")

    @functools.partial(
        pl.kernel,
        out_type=jax.ShapeDtypeStruct((seq, te, _NW, 8, 128), jnp.float32),
        mesh=mesh,
        compiler_params=pltpu.CompilerParams(use_tc_tiling_on_sc=False,
                                             needs_layout_passes=False),
        scratch_types=[
            pltpu.VMEM((seq, blk), jnp.int32),
            pltpu.VMEM((seq, emb), jnp.float32),
            pltpu.VMEM((blk, emb), jnp.float32),
            pltpu.VMEM((blk, emb), jnp.float32),
            pltpu.VMEM((te, 8, 128), jnp.float32),
            pltpu.VMEM((te, 8, 128), jnp.float32),
            pltpu.SemaphoreType.DMA,
            pltpu.SemaphoreType.DMA,
            pltpu.SemaphoreType.DMA,
            pltpu.SemaphoreType.DMA,
        ],
    )
    def emb_kernel(idx_hbm, tok_hbm, pos_hbm, out_hbm,
                   idx_all, pos_v, rows0, rows1, st0, st1,
                   g0, g1, o0, o1):
        wid = lax.axis_index("s") * _NC + lax.axis_index("c")
        pltpu.sync_copy(idx_hbm.at[:, wid], idx_all)
        pltpu.sync_copy(pos_hbm, pos_v)

        rows = (rows0, rows1)
        staged = (st0, st1)
        gsems = (g0, g1)
        osems = (o0, o1)
        lane = lax.iota(jnp.int32, _LANES)

        def gather_cp(s, buf):
            return (tok_hbm.at[idx_all.at[s]], rows[buf], gsems[buf])

        def out_cp(s, buf):
            return (staged[buf], out_hbm.at[s, :, wid], osems[buf])

        def start(args):
            pltpu.async_copy(*args)

        def wait(args):
            pltpu.make_async_copy(*args).wait()

        rids = [blk16 * _LANES + lane for blk16 in range(blk // _LANES)]

        def process(s, buf):
            rv = rows[buf]
            sv = staged[buf]
            srow = jnp.full((_LANES,), s, jnp.int32)

            def body(e, c):
                col = jnp.full((_LANES,), e, jnp.int32)
                pv = plsc.load_gather(pos_v, [srow, col])
                for blk16 in range(blk // _LANES):
                    vals = plsc.load_gather(rv, [rids[blk16], col]) + pv
                    sv[e // 8, e % 8, pl.ds(blk16 * _LANES, _LANES)] = vals
                return c

            lax.fori_loop(0, emb, body, 0)

        start(gather_cp(0, 0))

        def pair(k, c):
            s = 2 * k
            start(gather_cp(s + 1, 1))
            wait(gather_cp(s, 0))

            @pl.when(k > 0)
            def _():
                wait(out_cp(s - 2, 0))

            process(s, 0)
            start(out_cp(s, 0))

            @pl.when(k < seq // 2 - 1)
            def _():
                start(gather_cp(s + 2, 0))

            wait(gather_cp(s + 1, 1))

            @pl.when(k > 0)
            def _():
                wait(out_cp(s - 1, 1))

            process(s + 1, 1)
            start(out_cp(s + 1, 1))
            return c

        lax.fori_loop(0, seq // 2, pair, 0)
        wait(out_cp(seq - 2, 0))
        wait(out_cp(seq - 1, 1))

    return emb_kernel


def kernel(inputs, token_table, position_table):
    batch, seq = inputs.shape
    emb = token_table.shape[1]
    idx_t = inputs.astype(jnp.int32).T.reshape(seq, _NW, batch // _NW)
    f = _make_kernel(batch, seq, emb)
    out = f(idx_t, token_table, position_table)
    out = out.transpose(2, 4, 0, 1, 3).reshape(batch, seq, emb)
    return out


# speedup vs baseline: 1.8203x; 1.8203x over previous
import functools

import jax
import jax.numpy as jnp
from jax import lax
from jax.experimental import pallas as pl
from jax.experimental.pallas import tpu as pltpu
from jax.experimental.pallas import tpu_sc as plsc

_NC = 2
_NS = 16
_NW = _NC * _NS
_LANES = 16


@functools.cache
def _make_kernel(batch, seq, emb):
    blk = batch // _NW
    assert blk == 128 and emb % 8 == 0 and seq % 2 == 0
    te = emb // 8
    mesh = plsc.VectorSubcoreMesh(core_axis_name="c", subcore_axis_name="s")

    @functools.partial(
        pl.kernel,
        out_type=jax.ShapeDtypeStruct((seq, te, _NW, 8, 128), jnp.float32),
        mesh=mesh,
        compiler_params=pltpu.CompilerParams(use_tc_tiling_on_sc=False,
                                             needs_layout_passes=False),
        scratch_types=[
            pltpu.VMEM((seq, blk), jnp.int32),
            pltpu.VMEM((seq, emb), jnp.float32),
            pltpu.VMEM((blk, emb + 1), jnp.float32),
            pltpu.VMEM((blk, emb + 1), jnp.float32),
            pltpu.VMEM((te, 8, 128), jnp.float32),
            pltpu.VMEM((te, 8, 128), jnp.float32),
            pltpu.SemaphoreType.DMA,
            pltpu.SemaphoreType.DMA,
            pltpu.SemaphoreType.DMA,
            pltpu.SemaphoreType.DMA,
        ],
    )
    def emb_kernel(idx_hbm, tok_hbm, pos_hbm, out_hbm,
                   idx_all, pos_v, rows0, rows1, st0, st1,
                   g0, g1, o0, o1):
        wid = lax.axis_index("s") * _NC + lax.axis_index("c")
        pltpu.sync_copy(idx_hbm.at[:, wid], idx_all)
        pltpu.sync_copy(pos_hbm, pos_v)

        rows = (rows0, rows1)
        staged = (st0, st1)
        gsems = (g0, g1)
        osems = (o0, o1)
        lane = lax.iota(jnp.int32, _LANES)

        def gather_cp(s, buf):
            return (tok_hbm.at[idx_all.at[s]], rows[buf], gsems[buf])

        def out_cp(s, buf):
            return (staged[buf], out_hbm.at[s, :, wid], osems[buf])

        def start(args):
            pltpu.async_copy(*args)

        def wait(args):
            pltpu.make_async_copy(*args).wait()

        rids = [blk16 * _LANES + lane for blk16 in range(blk // _LANES)]

        def process(s, buf):
            rv = rows[buf]
            sv = staged[buf]
            srow = jnp.full((_LANES,), s, jnp.int32)

            def body(e, c):
                col = jnp.full((_LANES,), e, jnp.int32)
                pv = plsc.load_gather(pos_v, [srow, col])
                for blk16 in range(blk // _LANES):
                    vals = plsc.load_gather(rv, [rids[blk16], col]) + pv
                    sv[e // 8, e % 8, pl.ds(blk16 * _LANES, _LANES)] = vals
                return c

            lax.fori_loop(0, emb, body, 0)

        start(gather_cp(0, 0))

        def pair(k, c):
            s = 2 * k
            start(gather_cp(s + 1, 1))
            wait(gather_cp(s, 0))

            @pl.when(k > 0)
            def _():
                wait(out_cp(s - 2, 0))

            process(s, 0)
            start(out_cp(s, 0))

            @pl.when(k < seq // 2 - 1)
            def _():
                start(gather_cp(s + 2, 0))

            wait(gather_cp(s + 1, 1))

            @pl.when(k > 0)
            def _():
                wait(out_cp(s - 1, 1))

            process(s + 1, 1)
            start(out_cp(s + 1, 1))
            return c

        lax.fori_loop(0, seq // 2, pair, 0)
        wait(out_cp(seq - 2, 0))
        wait(out_cp(seq - 1, 1))

    return emb_kernel


def kernel(inputs, token_table, position_table):
    batch, seq = inputs.shape
    emb = token_table.shape[1]
    idx_t = inputs.astype(jnp.int32).T.reshape(seq, _NW, batch // _NW)
    tok65 = jnp.pad(token_table, ((0, 0), (0, 1)))
    f = _make_kernel(batch, seq, emb)
    out = f(idx_t, tok65, position_table)
    out = out.transpose(2, 4, 0, 1, 3).reshape(batch, seq, emb)
    return out
